# Initial kernel scaffold; baseline (speedup 1.0000x reference)
import jax
import jax.numpy as jnp
from jax.experimental import pallas as pl


def kernel(hidden_states, topk_weights, w_gate, w_up, w_down, lora_a, lora_b, scalings, topk_ids, lora_indices):
    raise NotImplementedError("write your pallas kernel here")


# speedup vs baseline: 3.1154x; 3.1154x over previous
import functools

import jax
import jax.numpy as jnp
from jax.experimental import pallas as pl
from jax.experimental.pallas import tpu as pltpu

T, K, E, D, F, L, R = 256, 2, 16, 1024, 1024, 4, 16
FB = 256
NFB = F // FB


def _moe_body(tw_ref, ids_ref, li_ref, scal_ref, x_ref, wg_ref, wu_ref,
              wd_ref, la_ref, lb_ref, o_ref):
    e = pl.program_id(0)
    fb = pl.program_id(1)

    @pl.when(jnp.logical_and(e == 0, fb == 0))
    def _init():
        o_ref[...] = jnp.zeros_like(o_ref)

    x = x_ref[...]
    xb = x.astype(jnp.bfloat16)

    tw = tw_ref[...]
    twn = tw / (jnp.sum(tw, axis=1, keepdims=True) + 1e-9)
    cvec = jnp.sum(jnp.where(ids_ref[...] == e, twn, 0.0),
                   axis=1, keepdims=True)

    wg = wg_ref[0].astype(jnp.bfloat16)
    wu = wu_ref[0].astype(jnp.bfloat16)
    g = jnp.dot(xb, wg, preferred_element_type=jnp.float32)
    u = jnp.dot(xb, wu, preferred_element_type=jnp.float32)
    h = (g / (1.0 + jnp.exp(-g))) * u
    wd = wd_ref[0].astype(jnp.bfloat16)
    o = jnp.dot(h.astype(jnp.bfloat16), wd,
                preferred_element_type=jnp.float32)
    acc = cvec * o

    @pl.when(fb == 0)
    def _lora():
        lio = jax.lax.broadcasted_iota(jnp.int32, (T, L), 1)
        sel = jnp.where(lio == li_ref[...], 1.0, 0.0) * scal_ref[...]
        a4 = la_ref[...].reshape(L * R, D).astype(jnp.bfloat16)
        za = jax.lax.dot_general(xb, a4, (((1,), (1,)), ((), ())),
                                 preferred_element_type=jnp.float32)
        delta = jnp.zeros((T, D), jnp.float32)
        for l in range(L):
            zs = za[:, l * R:(l + 1) * R] * sel[:, l:l + 1]
            bl = lb_ref[l, 0].astype(jnp.bfloat16)
            delta = delta + jax.lax.dot_general(
                zs.astype(jnp.bfloat16), bl, (((1,), (1,)), ((), ())),
                preferred_element_type=jnp.float32)
        o_ref[...] += cvec * delta

    o_ref[...] += acc


_moe_call = pl.pallas_call(
    _moe_body,
    grid=(E, NFB),
    in_specs=[
        pl.BlockSpec((T, K), lambda e, fb: (0, 0)),
        pl.BlockSpec((T, K), lambda e, fb: (0, 0)),
        pl.BlockSpec((T, 1), lambda e, fb: (0, 0)),
        pl.BlockSpec((1, L), lambda e, fb: (0, 0)),
        pl.BlockSpec((T, D), lambda e, fb: (0, 0)),
        pl.BlockSpec((1, D, FB), lambda e, fb: (e, 0, fb)),
        pl.BlockSpec((1, D, FB), lambda e, fb: (e, 0, fb)),
        pl.BlockSpec((1, FB, D), lambda e, fb: (e, fb, 0)),
        pl.BlockSpec((L, 1, R, D), lambda e, fb: (0, e, 0, 0)),
        pl.BlockSpec((L, 1, D, R), lambda e, fb: (0, e, 0, 0)),
    ],
    out_specs=pl.BlockSpec((T, D), lambda e, fb: (0, 0)),
    out_shape=jax.ShapeDtypeStruct((T, D), jnp.float32),
    compiler_params=pltpu.CompilerParams(
        dimension_semantics=("arbitrary", "arbitrary")),
)


def kernel(hidden_states, topk_weights, w_gate, w_up, w_down, lora_a,
           lora_b, scalings, topk_ids, lora_indices):
    li = lora_indices.reshape(T, 1)
    scal = scalings.reshape(1, L)
    return _moe_call(topk_weights, topk_ids, li, scal, hidden_states,
                     w_gate, w_up, w_down, lora_a, lora_b)


# speedup vs baseline: 4.2043x; 1.3495x over previous
import functools

import jax
import jax.numpy as jnp
from jax.experimental import pallas as pl
from jax.experimental.pallas import tpu as pltpu

T, K, E, D, F, L, R = 256, 2, 16, 1024, 1024, 4, 16


def _moe_body(tw_ref, ids_ref, li_ref, scal_ref, x_ref, wg_ref, wu_ref,
              wd_ref, la_ref, lb_ref, o_ref, xb_ref):
    e = pl.program_id(0)

    @pl.when(e == 0)
    def _init():
        o_ref[...] = jnp.zeros_like(o_ref)
        xb_ref[...] = x_ref[...].astype(jnp.bfloat16)

    xb = xb_ref[...]

    tw = tw_ref[...]
    twn = tw / (jnp.sum(tw, axis=1, keepdims=True) + 1e-9)
    cvec = jnp.sum(jnp.where(ids_ref[...] == e, twn, 0.0),
                   axis=1, keepdims=True)

    wg = wg_ref[0].astype(jnp.bfloat16)
    wu = wu_ref[0].astype(jnp.bfloat16)
    g = jnp.dot(xb, wg, preferred_element_type=jnp.float32)
    u = jnp.dot(xb, wu, preferred_element_type=jnp.float32)
    h = (g / (1.0 + jnp.exp(-g))) * u
    wd = wd_ref[0].astype(jnp.bfloat16)
    o = jnp.dot(h.astype(jnp.bfloat16), wd,
                preferred_element_type=jnp.float32)

    lio = jax.lax.broadcasted_iota(jnp.int32, (T, L), 1)
    sel = jnp.where(lio == li_ref[...], 1.0, 0.0) * scal_ref[...]
    a4 = la_ref[...].reshape(L * R, D).astype(jnp.bfloat16)
    za = jax.lax.dot_general(xb, a4, (((1,), (1,)), ((), ())),
                             preferred_element_type=jnp.float32)
    delta = o
    for l in range(L):
        zs = za[:, l * R:(l + 1) * R] * sel[:, l:l + 1]
        bl = lb_ref[l, 0].astype(jnp.bfloat16)
        delta = delta + jax.lax.dot_general(
            zs.astype(jnp.bfloat16), bl, (((1,), (1,)), ((), ())),
            preferred_element_type=jnp.float32)

    o_ref[...] += cvec * delta


_moe_call = pl.pallas_call(
    _moe_body,
    grid=(E,),
    in_specs=[
        pl.BlockSpec((T, K), lambda e: (0, 0)),
        pl.BlockSpec((T, K), lambda e: (0, 0)),
        pl.BlockSpec((T, 1), lambda e: (0, 0)),
        pl.BlockSpec((1, L), lambda e: (0, 0)),
        pl.BlockSpec((T, D), lambda e: (0, 0)),
        pl.BlockSpec((1, D, F), lambda e: (e, 0, 0)),
        pl.BlockSpec((1, D, F), lambda e: (e, 0, 0)),
        pl.BlockSpec((1, F, D), lambda e: (e, 0, 0)),
        pl.BlockSpec((L, 1, R, D), lambda e: (0, e, 0, 0)),
        pl.BlockSpec((L, 1, D, R), lambda e: (0, e, 0, 0)),
    ],
    out_specs=pl.BlockSpec((T, D), lambda e: (0, 0)),
    out_shape=jax.ShapeDtypeStruct((T, D), jnp.float32),
    scratch_shapes=[pltpu.VMEM((T, D), jnp.bfloat16)],
    compiler_params=pltpu.CompilerParams(
        dimension_semantics=("arbitrary",)),
)


def kernel(hidden_states, topk_weights, w_gate, w_up, w_down, lora_a,
           lora_b, scalings, topk_ids, lora_indices):
    li = lora_indices.reshape(T, 1)
    scal = scalings.reshape(1, L)
    return _moe_call(topk_weights, topk_ids, li, scal, hidden_states,
                     w_gate, w_up, w_down, lora_a, lora_b)


# speedup vs baseline: 4.4880x; 1.0675x over previous
import functools

import jax
import jax.numpy as jnp
from jax.experimental import pallas as pl
from jax.experimental.pallas import tpu as pltpu

T, K, E, D, F, L, R = 256, 2, 16, 1024, 1024, 4, 16


EPB = 2
GRID = E // EPB


def _moe_body(tw_ref, ids_ref, li_ref, scal_ref, x_ref, wg_ref, wu_ref,
              wd_ref, la_ref, lb_ref, o_ref, xb_ref):
    step = pl.program_id(0)

    @pl.when(step == 0)
    def _init():
        o_ref[...] = jnp.zeros_like(o_ref)
        xb_ref[...] = x_ref[...].astype(jnp.bfloat16)

    xb = xb_ref[...]

    tw = tw_ref[...]
    twn = tw / (jnp.sum(tw, axis=1, keepdims=True) + 1e-9)
    lio = jax.lax.broadcasted_iota(jnp.int32, (T, L), 1)
    sel = jnp.where(lio == li_ref[...], 1.0, 0.0) * scal_ref[...]

    acc = jnp.zeros((T, D), jnp.float32)
    for j in range(EPB):
        e = step * EPB + j
        cvec = jnp.sum(jnp.where(ids_ref[...] == e, twn, 0.0),
                       axis=1, keepdims=True)

        wg = wg_ref[j].astype(jnp.bfloat16)
        wu = wu_ref[j].astype(jnp.bfloat16)
        g = jnp.dot(xb, wg, preferred_element_type=jnp.float32)
        u = jnp.dot(xb, wu, preferred_element_type=jnp.float32)
        h = (g / (1.0 + jnp.exp(-g))) * u
        wd = wd_ref[j].astype(jnp.bfloat16)
        o = jnp.dot(h.astype(jnp.bfloat16), wd,
                    preferred_element_type=jnp.float32)

        a4 = la_ref[:, j].reshape(L * R, D).astype(jnp.bfloat16)
        za = jax.lax.dot_general(xb, a4, (((1,), (1,)), ((), ())),
                                 preferred_element_type=jnp.float32)
        delta = o
        for l in range(L):
            zs = za[:, l * R:(l + 1) * R] * sel[:, l:l + 1]
            bl = lb_ref[l, j].astype(jnp.bfloat16)
            delta = delta + jax.lax.dot_general(
                zs.astype(jnp.bfloat16), bl, (((1,), (1,)), ((), ())),
                preferred_element_type=jnp.float32)

        acc = acc + cvec * delta

    o_ref[...] += acc


_moe_call = pl.pallas_call(
    _moe_body,
    grid=(GRID,),
    in_specs=[
        pl.BlockSpec((T, K), lambda s: (0, 0)),
        pl.BlockSpec((T, K), lambda s: (0, 0)),
        pl.BlockSpec((T, 1), lambda s: (0, 0)),
        pl.BlockSpec((1, L), lambda s: (0, 0)),
        pl.BlockSpec((T, D), lambda s: (0, 0)),
        pl.BlockSpec((EPB, D, F), lambda s: (s, 0, 0)),
        pl.BlockSpec((EPB, D, F), lambda s: (s, 0, 0)),
        pl.BlockSpec((EPB, F, D), lambda s: (s, 0, 0)),
        pl.BlockSpec((L, EPB, R, D), lambda s: (0, s, 0, 0)),
        pl.BlockSpec((L, EPB, D, R), lambda s: (0, s, 0, 0)),
    ],
    out_specs=pl.BlockSpec((T, D), lambda e: (0, 0)),
    out_shape=jax.ShapeDtypeStruct((T, D), jnp.float32),
    scratch_shapes=[pltpu.VMEM((T, D), jnp.bfloat16)],
    compiler_params=pltpu.CompilerParams(
        dimension_semantics=("arbitrary",),
        vmem_limit_bytes=100 * 1024 * 1024),
)


def kernel(hidden_states, topk_weights, w_gate, w_up, w_down, lora_a,
           lora_b, scalings, topk_ids, lora_indices):
    li = lora_indices.reshape(T, 1)
    scal = scalings.reshape(1, L)
    return _moe_call(topk_weights, topk_ids, li, scal, hidden_states,
                     w_gate, w_up, w_down, lora_a, lora_b)
